# Initial kernel scaffold; baseline (speedup 1.0000x reference)
import jax
import jax.numpy as jnp
from jax.experimental import pallas as pl


def kernel(fea, params):
    raise NotImplementedError("write your pallas kernel here")


# speedup vs baseline: 1.0010x; 1.0010x over previous
import functools
import jax
import jax.numpy as jnp
from jax import lax
from jax.experimental import pallas as pl
from jax.experimental.pallas import tpu as pltpu

NPTS = 4096
KNN = 16
ENC_SPECS = [(3, 16, 24, 1), (24, 16, 32, 1), (32, 16, 48, 1), (48, 24, 48, 1),
             (48, 24, 64, 1), (64, 24, 64, 0), ('TD', 2048), (64, 24, 128, 1),
             (128, 32, 128, 0), ('TD', 1024), (128, 32, 256, 1), (256, 32, 256, 0),
             ('TD', 512), (256, 32, 512, 1), (512, 32, 512, 0), ('TD', 256),
             (512, 64, 1024, 1), (1024, 128, 1024, 0), ('TD', 128)]


def _gather_pts(points, idx):
    return jax.vmap(lambda p, i: p[i])(points, idx)


def _mlp(x, p, act):
    y = x @ p['W'] + p['b']
    y = y * p['gamma'] + p['beta']
    return jax.nn.leaky_relu(y, 0.2) if act else y


def _lfa(state, p):
    xyz, feature, ori_rel, nidx = state
    B, N, _ = feature.shape
    ori_feature = feature
    if ori_rel is None:
        sq = jnp.sum(xyz * xyz, -1)
        d2 = sq[:, :, None] + sq[:, None, :] - 2.0 * jnp.einsum('bnc,bmc->bnm', xyz, xyz)
        dists = jnp.sqrt(jnp.maximum(d2, 0.0))
        neg, nidx = jax.lax.top_k(-dists, KNN)
        rel_d = -neg
        feature = _gather_pts(feature, nidx)
        nxyz = _gather_pts(xyz, nidx)
        exp = jnp.broadcast_to(xyz[:, :, None, :], nxyz.shape)
        ori_rel = jnp.concatenate([rel_d[..., None], exp - nxyz, exp, nxyz], -1)
    else:
        feature = _gather_pts(feature, nidx)
    rel = _mlp(ori_rel.reshape(B, -1, 10), p['rel'], True).reshape(B, N, KNN, -1)
    feature = jnp.concatenate([feature, rel], -1)
    attn = jax.nn.softmax(feature @ p['attn_W'], axis=2)
    feature = jnp.sum(attn * feature, axis=2)
    feature = jax.nn.leaky_relu(_mlp(ori_feature, p['short'], False) + _mlp(feature, p['out'], False), 0.2)
    return (xyz, feature, ori_rel, nidx)


def _dec_out_body(x_ref, w0_ref, b0_ref, w1_ref, b1_ref, o_ref):
    x = x_ref[...]
    y = jax.nn.leaky_relu(x @ w0_ref[...] + b0_ref[...], 0.2)
    o_ref[...] = y @ w1_ref[...] + b1_ref[...]


def _fold(p):
    return p['W'] * p['gamma'][None, :], p['b'] * p['gamma'] + p['beta']


def _dec_out_pallas(feat, p0, p1):
    B, N, C = feat.shape
    w0, b0 = _fold(p0)
    w1, b1 = _fold(p1)
    x = feat.reshape(B * N, C)
    out = pl.pallas_call(
        _dec_out_body,
        out_shape=jax.ShapeDtypeStruct((B * N, w1.shape[1]), jnp.float32),
    )(x, w0, b0[None, :], w1, b1[None, :])
    return out.reshape(B, N, w1.shape[1])


def kernel(fea, params):
    B = fea.shape[0]
    xyz = fea[..., :3]
    state = (xyz, fea, None, None)
    li = 0
    for spec in ENC_SPECS:
        if spec[0] == 'TD':
            n = spec[1]
            state = (state[0][:, :n], state[1][:, :n], state[2], state[3])
        else:
            state = _lfa(state, params['enc'][li])
            li += 1
            if spec[3] == 0:
                state = (state[0], state[1], None, None)
    xyz_s, feat = state[0], state[1]
    feat = _mlp(feat, params['enc_out0'], True)
    feat = _mlp(feat, params['enc_out1'], False)
    feat = feat + jax.lax.stop_gradient(jnp.round(feat) - feat)
    state = (xyz_s, feat, None, None)
    state = _lfa(state, params['dec'][0])
    state = _lfa(state, params['dec'][1])
    feat = state[1]
    feat = _dec_out_pallas(feat, params['dec_out0'], params['dec_out1'])
    return feat.reshape(B, NPTS, 3)


# speedup vs baseline: 8.1600x; 8.1521x over previous
import functools
import jax
import jax.numpy as jnp
from jax import lax
from jax.experimental import pallas as pl
from jax.experimental.pallas import tpu as pltpu
from jax.experimental.pallas import tpu_sc as plsc

NPTS = 4096
KNN = 16
ENC_SPECS = [(3, 16, 24, 1), (24, 16, 32, 1), (32, 16, 48, 1), (48, 24, 48, 1),
             (48, 24, 64, 1), (64, 24, 64, 0), ('TD', 2048), (64, 24, 128, 1),
             (128, 32, 128, 0), ('TD', 1024), (128, 32, 256, 1), (256, 32, 256, 0),
             ('TD', 512), (256, 32, 512, 1), (512, 32, 512, 0), ('TD', 256),
             (512, 64, 1024, 1), (1024, 128, 1024, 0), ('TD', 128)]

_F32_INF = float('inf')


def _p16(c):
    return (c + 15) // 16 * 16


def _pad2(w, r, c):
    return jnp.pad(w, ((0, r - w.shape[0]), (0, c - w.shape[1])))


def _knn_body(q_ref, p_ref, sqq_ref, sqp_ref, nidx_ref, reld_ref, *, n, qb, k):
    q = q_ref[0]
    p = p_ref[0]
    dot = lax.dot_general(q, p, (((1,), (1,)), ((), ())),
                          preferred_element_type=jnp.float32)
    dist = jnp.sqrt(jnp.maximum((sqq_ref[0] + sqp_ref[0]) - 2.0 * dot, 0.0))
    iota = lax.broadcasted_iota(jnp.int32, (qb, n), 1)
    nidx_cols = []
    reld_cols = []
    for _ in range(k):
        m = jnp.min(dist, axis=1, keepdims=True)
        idx = jnp.min(jnp.where(dist == m, iota, n), axis=1, keepdims=True)
        nidx_cols.append(idx)
        reld_cols.append(m)
        dist = jnp.where(iota == idx, _F32_INF, dist)
    nidx_ref[0] = jnp.concatenate(nidx_cols, axis=1)
    reld_ref[0] = jnp.concatenate(reld_cols, axis=1)


def _knn(xyz_pad, sq):
    B, N, _ = xyz_pad.shape
    sqq = sq[:, :, None]
    sqp = sq[:, None, :]
    QB = min(N, 128)
    grid = (B, N // QB)
    kern = functools.partial(_knn_body, n=N, qb=QB, k=KNN)
    return pl.pallas_call(
        kern,
        grid=grid,
        in_specs=[
            pl.BlockSpec((1, QB, 16), lambda b, i: (b, i, 0)),
            pl.BlockSpec((1, N, 16), lambda b, i: (b, 0, 0)),
            pl.BlockSpec((1, QB, 1), lambda b, i: (b, i, 0)),
            pl.BlockSpec((1, 1, N), lambda b, i: (b, 0, 0)),
        ],
        out_specs=[
            pl.BlockSpec((1, QB, KNN), lambda b, i: (b, i, 0)),
            pl.BlockSpec((1, QB, KNN), lambda b, i: (b, i, 0)),
        ],
        out_shape=[
            jax.ShapeDtypeStruct((B, N, KNN), jnp.int32),
            jax.ShapeDtypeStruct((B, N, KNN), jnp.float32),
        ],
    )(xyz_pad, xyz_pad, sqq, sqp)


@functools.lru_cache(maxsize=None)
def _make_sc_gather(R, D, M):
    info = plsc.get_sparse_core_info()
    NC, NS = info.num_cores, info.num_subcores
    NW = NC * NS
    cnt = M // NW
    chunk = min(cnt, 128 if D <= 512 else 64)
    nloop = cnt // chunk
    mesh = plsc.VectorSubcoreMesh(core_axis_name="c", subcore_axis_name="s")

    @functools.partial(
        pl.kernel, mesh=mesh,
        out_type=jax.ShapeDtypeStruct((M, D), jnp.float32),
        compiler_params=pltpu.CompilerParams(use_tc_tiling_on_sc=False),
        scratch_types=[
            pltpu.VMEM((chunk,), jnp.int32),
            pltpu.VMEM((chunk, D), jnp.float32),
            pltpu.SemaphoreType.DMA,
        ],
    )
    def gather(table_hbm, idx_hbm, out_hbm, idx_v, rows_v, sem):
        wid = lax.axis_index("s") * NC + lax.axis_index("c")
        base = wid * cnt

        def body(i, carry):
            off = pl.multiple_of(base + i * chunk, 8)
            pltpu.sync_copy(idx_hbm.at[pl.ds(off, chunk)], idx_v)
            pltpu.async_copy(table_hbm.at[idx_v], rows_v, sem).wait()
            pltpu.sync_copy(rows_v, out_hbm.at[pl.ds(off, chunk)])
            return carry

        lax.fori_loop(0, nloop, body, 0)

    return gather


def _sc_gather(table, idx_flat):
    R, D = table.shape
    M = idx_flat.shape[0]
    return _make_sc_gather(R, D, M)(table, idx_flat)


def _lfa_body(feat_ref, fg_ref, reld_ref, nxyz_ref, xyz_ref,
              wrel_ref, brel_ref, grel_ref, trel_ref,
              attnw_ref,
              wo_ref, bo_ref, go_ref, to_ref,
              ws_ref, bs_ref, gs_ref, ts_ref,
              out_ref, *, p, k, cin, cout):
    fg = fg_ref[...]
    reld = reld_ref[...]
    nx = nxyz_ref[...][:, :3]
    xq = xyz_ref[...][:, :3]
    feat = feat_ref[...]

    f = fg[:, :cin] if fg.shape[1] != cin else fg
    xq3 = jnp.broadcast_to(xq.reshape(p, 1, 3), (p, k, 3)).reshape(p * k, 3)
    ori10 = jnp.concatenate([reld, xq3 - nx, xq3, nx], axis=1)
    rel = jnp.dot(ori10, wrel_ref[...]) + brel_ref[...]
    rel = rel * grel_ref[...] + trel_ref[...]
    rel = jnp.where(rel >= 0, rel, 0.2 * rel)

    cat = jnp.concatenate([f, rel], axis=1)
    c = cat.shape[1]

    def sumk(v):
        s = v[:, 0:1]
        for i in range(1, k):
            s = s + v[:, i:i + 1]
        return s

    logits = jnp.dot(cat, attnw_ref[...]).reshape(p, k, c)
    m = jnp.max(logits, axis=1, keepdims=True)
    e = jnp.exp(logits - m)
    a = e / sumk(e)
    pooled = sumk(a * cat.reshape(p, k, c))[:, 0]

    yo = jnp.dot(pooled, wo_ref[...]) + bo_ref[...]
    yo = yo * go_ref[...] + to_ref[...]
    ys = jnp.dot(feat, ws_ref[...]) + bs_ref[...]
    ys = ys * gs_ref[...] + ts_ref[...]
    y = ys + yo
    y = jnp.where(y >= 0, y, 0.2 * y)
    if cout != out_ref.shape[1]:
        y = jnp.concatenate(
            [y, jnp.zeros((p, out_ref.shape[1] - cout), jnp.float32)], axis=1)
    out_ref[...] = y


def _lfa_pallas(feat, fg, reld_flat, nxyz, xyz_pad, wp, cin, relc, cout, first):
    BN, cinp = feat.shape
    coutp = _p16(cout)
    P = 64 if cinp >= 512 else 128
    grid = (BN // P,)
    kern = functools.partial(_lfa_body, p=P, k=KNN, cin=cin, cout=cout)
    wspec = lambda w: pl.BlockSpec(w.shape, lambda i: tuple(0 for _ in w.shape))
    return pl.pallas_call(
        kern,
        grid=grid,
        in_specs=[
            pl.BlockSpec((P, cinp), lambda i: (i, 0)),
            pl.BlockSpec((P * KNN, cinp), lambda i: (i, 0)),
            pl.BlockSpec((P * KNN, 1), lambda i: (i, 0)),
            pl.BlockSpec((P * KNN, 16), lambda i: (i, 0)),
            pl.BlockSpec((P, 16), lambda i: (i, 0)),
        ] + [wspec(wp[n]) for n in _LFA_WNAMES],
        out_specs=pl.BlockSpec((P, coutp), lambda i: (i, 0)),
        out_shape=jax.ShapeDtypeStruct((BN, coutp), jnp.float32),
    )(feat, fg, reld_flat, nxyz, xyz_pad, *[wp[n] for n in _LFA_WNAMES])


_LFA_WNAMES = ['wrel', 'brel', 'grel', 'trel', 'attnw',
               'wo', 'bo', 'go', 'to', 'ws', 'bs', 'gs', 'ts']


def _prep_lfa_weights(p, cin, relc, cout):
    cinp = _p16(cin)
    row = lambda v: v[None, :]
    return {
        'wrel': p['rel']['W'],
        'brel': row(p['rel']['b']), 'grel': row(p['rel']['gamma']),
        'trel': row(p['rel']['beta']),
        'attnw': p['attn_W'],
        'wo': p['out']['W'], 'bo': row(p['out']['b']),
        'go': row(p['out']['gamma']), 'to': row(p['out']['beta']),
        'ws': _pad2(p['short']['W'], cinp, cout), 'bs': row(p['short']['b']),
        'gs': row(p['short']['gamma']), 'ts': row(p['short']['beta']),
    }


def _mlp2_body(x_ref, w0_ref, b0_ref, g0_ref, t0_ref,
               w1_ref, b1_ref, g1_ref, t1_ref, o_ref, *, do_round):
    x = x_ref[...]
    y = (jnp.dot(x, w0_ref[...]) + b0_ref[...]) * g0_ref[...] + t0_ref[...]
    y = jnp.where(y >= 0, y, 0.2 * y)
    z = (jnp.dot(y, w1_ref[...]) + b1_ref[...]) * g1_ref[...] + t1_ref[...]
    o_ref[...] = jnp.round(z) if do_round else z


def _mlp2_pallas(x, p0, p1, do_round):
    row = lambda v: v[None, :]
    return pl.pallas_call(
        functools.partial(_mlp2_body, do_round=do_round),
        out_shape=jax.ShapeDtypeStruct((x.shape[0], p1['W'].shape[1]), jnp.float32),
    )(x, p0['W'], row(p0['b']), row(p0['gamma']), row(p0['beta']),
      p1['W'], row(p1['b']), row(p1['gamma']), row(p1['beta']))


def _group_knn(xyz_pad, xyz, B, N):
    sq = jnp.sum(xyz * xyz, -1)
    nidx, rel_d = _knn(xyz_pad, sq)
    flat_idx = (nidx + (jnp.arange(B, dtype=jnp.int32) * N)[:, None, None]).reshape(-1)
    return flat_idx, rel_d.reshape(-1, 1)


def kernel(fea, params):
    B, N, _ = fea.shape
    xyz = fea[..., :3]
    xyz_pad = jnp.pad(xyz, ((0, 0), (0, 0), (0, 13)))
    feat = xyz_pad
    cin_cur = 3

    flat_idx = None
    reld_flat = None
    nxyz = None
    li = 0
    for spec in ENC_SPECS:
        if spec[0] == 'TD':
            n = spec[1]
            xyz_pad = xyz_pad[:, :n]
            xyz = xyz[:, :n]
            feat = feat.reshape(B, N, -1)[:, :n].reshape(B * n, -1)
            N = n
            continue
        cin, relc, cout, keep = spec
        if flat_idx is None:
            flat_idx, reld_flat = _group_knn(xyz_pad, xyz, B, N)
            nxyz = _sc_gather(xyz_pad.reshape(B * N, 16), flat_idx)
            feat = feat.reshape(B * N, -1)
        if cin == 3:
            fg = nxyz
        else:
            fg = _sc_gather(feat, flat_idx)
        wp = _prep_lfa_weights(params['enc'][li], cin, relc, cout)
        feat = _lfa_pallas(feat, fg, reld_flat, nxyz, xyz_pad.reshape(B * N, 16),
                           wp, cin, relc, cout, cin == 3)
        li += 1
        if keep == 0:
            flat_idx = None

    feat = _mlp2_pallas(feat, params['enc_out0'], params['enc_out1'], True)

    flat_idx, reld_flat = _group_knn(xyz_pad, xyz, B, N)
    nxyz = _sc_gather(xyz_pad.reshape(B * N, 16), flat_idx)
    for dp in params['dec']:
        fg = _sc_gather(feat, flat_idx)
        wp = _prep_lfa_weights(dp, 1024, 128, 1024)
        feat = _lfa_pallas(feat, fg, reld_flat, nxyz, xyz_pad.reshape(B * N, 16),
                           wp, 1024, 128, 1024, False)

    out = _mlp2_pallas(feat, params['dec_out0'], params['dec_out1'], False)
    return out.reshape(B, NPTS, 3)
